# baseline (device time: 12919 ns/iter reference)
import jax
import jax.numpy as jnp
from jax import lax
from jax.experimental import pallas as pl
from jax.experimental.pallas import tpu as pltpu

N_DEV = 8
BLK = 128

SEND_ORDER = (2, 6, 3, 5, 1, 7, 4)
RECV_ORDER = (4, 1, 7, 3, 5, 2, 6)


def kernel(x, w_mat):
    m, k_per = x.shape
    k, n = w_mat.shape
    assert m == N_DEV * BLK and k_per == BLK and k == N_DEV * BLK

    def body(x_ref, w_hbm, out_ref, sendbuf_ref, comm_ref, w_ref,
             out_vmem_ref, send_sems, recv_sems, w_sem, out_sems):
        my = lax.axis_index("i")

        barrier_sem = pltpu.get_barrier_semaphore()
        for off in range(1, N_DEV):
            tgt = lax.rem(my + off, N_DEV)
            pl.semaphore_signal(
                barrier_sem, inc=1,
                device_id=(tgt,), device_id_type=pl.DeviceIdType.MESH,
            )

        w_copy = pltpu.make_async_copy(w_hbm, w_ref, w_sem)
        w_copy.start()

        for off in SEND_ORDER:
            dst = lax.rem(my + off, N_DEV)
            sendbuf_ref[off - 1, :, :] = x_ref[
                pl.ds(dst * BLK, BLK), :
            ].astype(jnp.bfloat16)

        pl.semaphore_wait(barrier_sem, N_DEV - 1)

        rdmas = {}
        for off in SEND_ORDER:
            dst = lax.rem(my + off, N_DEV)
            rdma = pltpu.make_async_remote_copy(
                src_ref=sendbuf_ref.at[off - 1],
                dst_ref=comm_ref.at[off - 1],
                send_sem=send_sems.at[off - 1],
                recv_sem=recv_sems.at[off - 1],
                device_id=(dst,),
                device_id_type=pl.DeviceIdType.MESH,
            )
            rdma.start()
            rdmas[off] = rdma

        w_copy.wait()
        acc = jnp.dot(
            x_ref[pl.ds(my * BLK, BLK), :],
            w_ref[pl.ds(my * BLK, BLK), :],
            preferred_element_type=jnp.float32,
        )
        for off in RECV_ORDER:
            rdmas[off].wait_recv()
            src = lax.rem(my + N_DEV - off, N_DEV)
            acc += jnp.dot(
                comm_ref[off - 1],
                w_ref[pl.ds(src * BLK, BLK), :],
                preferred_element_type=jnp.float32,
            )

        c2 = 2.0 * 0.7978845608028654
        half = n // 2
        out_copies = []
        for h in range(2):
            a = acc[:, h * half:(h + 1) * half]
            z2 = c2 * (a + 0.044715 * a * a * a)
            out_vmem_ref[:, h * half:(h + 1) * half] = a / (1.0 + jnp.exp(-z2))
            cp = pltpu.make_async_copy(
                out_vmem_ref.at[:, pl.ds(h * half, half)],
                out_ref.at[:, pl.ds(h * half, half)],
                out_sems.at[h],
            )
            cp.start()
            out_copies.append(cp)

        for off in SEND_ORDER:
            rdmas[off].wait_send()
        for cp in out_copies:
            cp.wait()

    return pl.pallas_call(
        body,
        out_shape=jax.ShapeDtypeStruct((BLK, n), jnp.float32),
        in_specs=[
            pl.BlockSpec(memory_space=pltpu.VMEM),
            pl.BlockSpec(memory_space=pltpu.MemorySpace.HBM),
        ],
        out_specs=pl.BlockSpec(memory_space=pltpu.MemorySpace.HBM),
        scratch_shapes=[
            pltpu.VMEM((N_DEV - 1, BLK, BLK), jnp.bfloat16),
            pltpu.VMEM((N_DEV - 1, BLK, BLK), jnp.bfloat16),
            pltpu.VMEM((N_DEV * BLK, n), w_mat.dtype),
            pltpu.VMEM((BLK, n), jnp.float32),
            pltpu.SemaphoreType.DMA((N_DEV - 1,)),
            pltpu.SemaphoreType.DMA((N_DEV - 1,)),
            pltpu.SemaphoreType.DMA,
            pltpu.SemaphoreType.DMA((2,)),
        ],
        compiler_params=pltpu.CompilerParams(collective_id=0),
    )(x, w_mat)


# device time: 11908 ns/iter; 1.0849x vs baseline; 1.0849x over previous
import jax
import jax.numpy as jnp
from jax import lax
from jax.experimental import pallas as pl
from jax.experimental.pallas import tpu as pltpu

N_DEV = 8
BLK = 128

SEND_ORDER = (2, 6, 3, 5, 1, 7, 4)
RECV_ORDER = (4, 1, 7, 3, 5, 2, 6)


def kernel(x, w_mat):
    m, k_per = x.shape
    k, n = w_mat.shape
    assert m == N_DEV * BLK and k_per == BLK and k == N_DEV * BLK

    def body(x_ref, w_hbm, out_ref, comm_ref, w_ref, out_vmem_ref,
             send_sems, recv_sems, w_sem, out_sems):
        my = lax.axis_index("i")

        barrier_sem = pltpu.get_barrier_semaphore()
        for off in range(1, N_DEV):
            tgt = lax.rem(my + off, N_DEV)
            pl.semaphore_signal(
                barrier_sem, inc=1,
                device_id=(tgt,), device_id_type=pl.DeviceIdType.MESH,
            )

        w_copy = pltpu.make_async_copy(w_hbm, w_ref, w_sem)
        w_copy.start()

        pl.semaphore_wait(barrier_sem, N_DEV - 1)

        rdmas = {}
        for off in SEND_ORDER:
            dst = lax.rem(my + off, N_DEV)
            rdma = pltpu.make_async_remote_copy(
                src_ref=x_ref.at[pl.ds(dst * BLK, BLK), :],
                dst_ref=comm_ref.at[off - 1],
                send_sem=send_sems.at[off - 1],
                recv_sem=recv_sems.at[off - 1],
                device_id=(dst,),
                device_id_type=pl.DeviceIdType.MESH,
            )
            rdma.start()
            rdmas[off] = rdma

        w_copy.wait()
        acc = jnp.dot(
            x_ref[pl.ds(my * BLK, BLK), :],
            w_ref[pl.ds(my * BLK, BLK), :],
            preferred_element_type=jnp.float32,
        )
        for off in RECV_ORDER:
            rdmas[off].wait_recv()
            src = lax.rem(my + N_DEV - off, N_DEV)
            acc += jnp.dot(
                comm_ref[off - 1],
                w_ref[pl.ds(src * BLK, BLK), :],
                preferred_element_type=jnp.float32,
            )

        c2 = 2.0 * 0.7978845608028654
        half = n // 2
        out_copies = []
        for h in range(2):
            a = acc[:, h * half:(h + 1) * half]
            z2 = c2 * (a + 0.044715 * a * a * a)
            out_vmem_ref[:, h * half:(h + 1) * half] = a / (1.0 + jnp.exp(-z2))
            cp = pltpu.make_async_copy(
                out_vmem_ref.at[:, pl.ds(h * half, half)],
                out_ref.at[:, pl.ds(h * half, half)],
                out_sems.at[h],
            )
            cp.start()
            out_copies.append(cp)

        for off in SEND_ORDER:
            rdmas[off].wait_send()
        for cp in out_copies:
            cp.wait()

    return pl.pallas_call(
        body,
        out_shape=jax.ShapeDtypeStruct((BLK, n), jnp.float32),
        in_specs=[
            pl.BlockSpec(memory_space=pltpu.VMEM),
            pl.BlockSpec(memory_space=pltpu.MemorySpace.HBM),
        ],
        out_specs=pl.BlockSpec(memory_space=pltpu.MemorySpace.HBM),
        scratch_shapes=[
            pltpu.VMEM((N_DEV - 1, BLK, BLK), jnp.bfloat16),
            pltpu.VMEM((N_DEV * BLK, n), jnp.bfloat16),
            pltpu.VMEM((BLK, n), jnp.float32),
            pltpu.SemaphoreType.DMA((N_DEV - 1,)),
            pltpu.SemaphoreType.DMA((N_DEV - 1,)),
            pltpu.SemaphoreType.DMA,
            pltpu.SemaphoreType.DMA((2,)),
        ],
        compiler_params=pltpu.CompilerParams(collective_id=0),
    )(x.astype(jnp.bfloat16), w_mat.astype(jnp.bfloat16))
